# baseline (speedup 1.0000x reference)
import functools

import jax
import jax.numpy as jnp
import numpy as np
from jax import lax
from jax.experimental import pallas as pl
from jax.experimental.pallas import tpu as pltpu
from jax.experimental.pallas import tpu_sc as plsc

_INPUT_DIM = 3
_NUM_LEVELS = 16
_LEVEL_DIM = 2
_BASE_RES = 16
_MAX_PARAMS = 2 ** 19
_B = 262144
_OUT_DIM = _NUM_LEVELS * _LEVEL_DIM

_P1 = np.int32(np.int64(2654435761) - (1 << 32))
_P2 = np.int32(805459861)

_NC, _NS = 2, 16
_NW = _NC * _NS
_PPW = _B // _NW
_C = 128
_NCH = _PPW // _C
_NG = _C // 16


def _level_offsets():
    offs, o = [0], 0
    for i in range(_NUM_LEVELS):
        res = _BASE_RES * (2 ** i)
        o += min(_MAX_PARAMS, (res + 1) ** _INPUT_DIM)
        offs.append(o)
    return offs


_OFFS = _level_offsets()
_TABLE_ROWS = _OFFS[-1]
_FLAT = _TABLE_ROWS * _LEVEL_DIM

_mesh = plsc.VectorSubcoreMesh(core_axis_name="c", subcore_axis_name="s")


@functools.partial(
    pl.kernel,
    out_type=jax.ShapeDtypeStruct((_B, _OUT_DIM), jnp.float32),
    mesh=_mesh,
    compiler_params=pltpu.CompilerParams(needs_layout_passes=False,
                                         use_tc_tiling_on_sc=False),
    scratch_types=[
        pltpu.VMEM((_PPW, _INPUT_DIM), jnp.float32),
        pltpu.VMEM((_INPUT_DIM * _PPW,), jnp.float32),
        pltpu.VMEM((16, _C), jnp.int32),
        pltpu.VMEM((16, _C), jnp.int32),
        pltpu.VMEM((8, _C), jnp.float32),
        pltpu.VMEM((8, _C), jnp.float32),
        pltpu.VMEM((16, _C), jnp.float32),
        pltpu.VMEM((16, _C), jnp.float32),
        pltpu.VMEM((_C, _OUT_DIM), jnp.float32),
        pltpu.SemaphoreType.DMA,
        pltpu.SemaphoreType.DMA,
    ],
)
def _encode(in_hbm, emb_hbm, out_hbm, xin, xs, idx0, idx1,
            w0, w1, rows0, rows1, out_buf, semA, semB):
    idxb, wb, rowsb, sems = ((idx0, idx1), (w0, w1), (rows0, rows1),
                             (semA, semB))
    emb_flat = emb_hbm

    cid = lax.axis_index("c")
    sid = lax.axis_index("s")
    wid = sid * _NC + cid
    base = wid * _PPW

    pltpu.sync_copy(in_hbm.at[pl.ds(base, _PPW), :], xin)

    iota = lax.iota(jnp.int32, 16)

    def deint(g, carry):
        pv = g * 16 + iota
        for d in range(_INPUT_DIM):
            v = plsc.load_gather(xin, [pv, jnp.full((16,), d, jnp.int32)])
            x = jnp.clip((v + 1.0) * 0.5, 0.0, 1.0)
            xs[pl.ds(d * _PPW + g * 16, 16)] = x
        return carry

    lax.fori_loop(0, _PPW // 16, deint, 0)

    def level_const(l):
        res = _BASE_RES * (2 ** l)
        ts = _OFFS[l + 1] - _OFFS[l]
        return res, ts, _OFFS[l], (res + 1) ** _INPUT_DIM > ts

    def make_comp(cbase, l):
        res, ts, off, use_hash = level_const(l)
        resf = float(res)
        par = l % 2
        idx_buf, w_buf = idxb[par], wb[par]

        def comp(g, c2):
            p0 = cbase + g * 16
            ii, fr = [], []
            for d in range(_INPUT_DIM):
                x = xs[pl.ds(d * _PPW + p0, 16)]
                pos = x * resf
                i = jnp.minimum(pos.astype(jnp.int32), jnp.int32(res - 1))
                ii.append(i)
                fr.append(pos - i.astype(jnp.float32))
            if use_hash:
                lo = [ii[0], ii[1] * _P1, ii[2] * _P2]
                hi = [ii[0] + 1, (ii[1] + 1) * _P1, (ii[2] + 1) * _P2]
            else:
                s1, s2 = res + 1, (res + 1) ** 2
                lo = [ii[0], ii[1] * jnp.int32(s1), ii[2] * jnp.int32(s2)]
                hi = [ii[0] + 1, lo[1] + s1, lo[2] + s2]
            t = [1.0 - fr[d] for d in range(_INPUT_DIM)]
            q = [t[0] * t[1], fr[0] * t[1], t[0] * fr[1], fr[0] * fr[1]]
            for c in range(8):
                b0, b1, b2 = c & 1, (c >> 1) & 1, (c >> 2) & 1
                e0 = hi[0] if b0 else lo[0]
                e1 = hi[1] if b1 else lo[1]
                e2 = hi[2] if b2 else lo[2]
                if use_hash:
                    idx = ((e0 ^ e1) ^ e2) & jnp.int32(ts - 1)
                else:
                    idx = e0 + e1 + e2
                flat = idx * 2 + jnp.int32(2 * off)
                idx_buf[2 * c, pl.ds(g * 16, 16)] = flat
                idx_buf[2 * c + 1, pl.ds(g * 16, 16)] = flat + 1
                w = q[b0 + 2 * b1] * (fr[2] if b2 else t[2])
                w_buf[c, pl.ds(g * 16, 16)] = w
            return c2

        lax.fori_loop(0, _NG, comp, 0)
        par_sem = sems[par]
        return [pltpu.async_copy(emb_flat.at[idxb[par].at[j]],
                                 rowsb[par].at[j], par_sem)
                for j in range(16)]

    def drain_acc(cps, l):
        for cp in cps:
            cp.wait()
        par = l % 2
        w_buf, rows_buf = wb[par], rowsb[par]
        col0 = jnp.full((16,), 2 * l, jnp.int32)
        col1 = jnp.full((16,), 2 * l + 1, jnp.int32)

        def acc(g, c2):
            pv = g * 16 + iota
            a0 = jnp.zeros((16,), jnp.float32)
            a1 = jnp.zeros((16,), jnp.float32)
            for c in range(8):
                w = w_buf[c, pl.ds(g * 16, 16)]
                f0 = rows_buf[2 * c, pl.ds(g * 16, 16)]
                f1 = rows_buf[2 * c + 1, pl.ds(g * 16, 16)]
                a0 = a0 + w * f0
                a1 = a1 + w * f1
            plsc.store_scatter(out_buf, [pv, col0], a0)
            plsc.store_scatter(out_buf, [pv, col1], a1)
            return c2

        lax.fori_loop(0, _NG, acc, 0)

    def chunk_body(ch, carry):
        cbase = ch * _C
        pend = make_comp(cbase, 0)
        for l in range(1, _NUM_LEVELS):
            nxt = make_comp(cbase, l)
            drain_acc(pend, l - 1)
            pend = nxt
        drain_acc(pend, _NUM_LEVELS - 1)
        pltpu.sync_copy(out_buf, out_hbm.at[pl.ds(base + cbase, _C), :])
        return carry

    lax.fori_loop(0, _NCH, chunk_body, 0)


def kernel(inputs, embeddings):
    return _encode(inputs, embeddings.reshape(-1))


# speedup vs baseline: 4.7643x; 4.7643x over previous
import functools

import jax
import jax.numpy as jnp
import numpy as np
from jax import lax
from jax.experimental import pallas as pl
from jax.experimental.pallas import tpu as pltpu
from jax.experimental.pallas import tpu_sc as plsc

_INPUT_DIM = 3
_NUM_LEVELS = 16
_LEVEL_DIM = 2
_BASE_RES = 16
_MAX_PARAMS = 2 ** 19
_B = 262144
_OUT_DIM = _NUM_LEVELS * _LEVEL_DIM

_P1 = np.int32(np.int64(2654435761) - (1 << 32))
_P2 = np.int32(805459861)

_NC, _NS = 2, 16
_NW = _NC * _NS
_PPW = _B // _NW
_C = 128
_NCH = _PPW // _C
_NG = _C // 16


def _level_offsets():
    offs, o = [0], 0
    for i in range(_NUM_LEVELS):
        res = _BASE_RES * (2 ** i)
        o += min(_MAX_PARAMS, (res + 1) ** _INPUT_DIM)
        offs.append(o)
    return offs


_OFFS = _level_offsets()
_TABLE_ROWS = _OFFS[-1]

_mesh = plsc.VectorSubcoreMesh(core_axis_name="c", subcore_axis_name="s")


@functools.partial(
    pl.kernel,
    out_type=jax.ShapeDtypeStruct((_B, _OUT_DIM), jnp.float32),
    mesh=_mesh,
    compiler_params=pltpu.CompilerParams(needs_layout_passes=False,
                                         use_tc_tiling_on_sc=False),
    scratch_types=[
        pltpu.VMEM((_PPW, _INPUT_DIM), jnp.float32),
        pltpu.VMEM((_INPUT_DIM * _PPW,), jnp.float32),
        pltpu.VMEM((8, _C), jnp.int32),
        pltpu.VMEM((8, _C), jnp.int32),
        pltpu.VMEM((8, _C), jnp.float32),
        pltpu.VMEM((8, _C), jnp.float32),
        pltpu.VMEM((8, _C), jnp.int32),
        pltpu.VMEM((8, _C), jnp.int32),
        pltpu.VMEM((_C, _OUT_DIM), jnp.float32),
        pltpu.SemaphoreType.DMA,
        pltpu.SemaphoreType.DMA,
    ],
)
def _encode(in_hbm, emb_hbm, out_hbm, xin, xs, idx0, idx1,
            w0, w1, rows0, rows1, out_buf, semA, semB):
    idxb, wb, rowsb, sems = ((idx0, idx1), (w0, w1), (rows0, rows1),
                             (semA, semB))

    cid = lax.axis_index("c")
    sid = lax.axis_index("s")
    wid = sid * _NC + cid
    base = wid * _PPW

    pltpu.sync_copy(in_hbm.at[pl.ds(base, _PPW), :], xin)

    iota = lax.iota(jnp.int32, 16)

    def deint(g, carry):
        pv = g * 16 + iota
        for d in range(_INPUT_DIM):
            v = plsc.load_gather(xin, [pv, jnp.full((16,), d, jnp.int32)])
            x = jnp.clip((v + 1.0) * 0.5, 0.0, 1.0)
            xs[pl.ds(d * _PPW + g * 16, 16)] = x
        return carry

    lax.fori_loop(0, _PPW // 16, deint, 0)

    def level_const(l):
        res = _BASE_RES * (2 ** l)
        ts = _OFFS[l + 1] - _OFFS[l]
        return res, ts, _OFFS[l], (res + 1) ** _INPUT_DIM > ts

    def make_comp(cbase, l):
        res, ts, off, use_hash = level_const(l)
        resf = float(res)
        par = l % 2
        idx_buf, w_buf = idxb[par], wb[par]

        def comp(g, c2):
            p0 = cbase + g * 16
            ii, fr = [], []
            for d in range(_INPUT_DIM):
                x = xs[pl.ds(d * _PPW + p0, 16)]
                pos = x * resf
                i = jnp.minimum(pos.astype(jnp.int32), jnp.int32(res - 1))
                ii.append(i)
                fr.append(pos - i.astype(jnp.float32))
            if use_hash:
                lo = [ii[0], ii[1] * _P1, ii[2] * _P2]
                hi = [ii[0] + 1, (ii[1] + 1) * _P1, (ii[2] + 1) * _P2]
            else:
                s1, s2 = res + 1, (res + 1) ** 2
                lo = [ii[0], ii[1] * jnp.int32(s1), ii[2] * jnp.int32(s2)]
                hi = [ii[0] + 1, lo[1] + s1, lo[2] + s2]
            t = [1.0 - fr[d] for d in range(_INPUT_DIM)]
            q = [t[0] * t[1], fr[0] * t[1], t[0] * fr[1], fr[0] * fr[1]]
            for c in range(8):
                b0, b1, b2 = c & 1, (c >> 1) & 1, (c >> 2) & 1
                e0 = hi[0] if b0 else lo[0]
                e1 = hi[1] if b1 else lo[1]
                e2 = hi[2] if b2 else lo[2]
                if use_hash:
                    idx = ((e0 ^ e1) ^ e2) & jnp.int32(ts - 1)
                else:
                    idx = e0 + e1 + e2
                idx_buf[c, pl.ds(g * 16, 16)] = idx + jnp.int32(off)
                w = q[b0 + 2 * b1] * (fr[2] if b2 else t[2])
                w_buf[c, pl.ds(g * 16, 16)] = w
            return c2

        lax.fori_loop(0, _NG, comp, 0)
        par_sem = sems[par]
        return [pltpu.async_copy(emb_hbm.at[idxb[par].at[c]],
                                 rowsb[par].at[c], par_sem)
                for c in range(8)]

    _HI = np.int32(np.int64(0xFFFF0000) - (1 << 32))

    def drain_acc(cps, l):
        for cp in cps:
            cp.wait()
        par = l % 2
        w_buf, rows_buf = wb[par], rowsb[par]
        col0 = jnp.full((16,), 2 * l, jnp.int32)
        col1 = jnp.full((16,), 2 * l + 1, jnp.int32)

        def acc(g, c2):
            pv = g * 16 + iota
            a0 = jnp.zeros((16,), jnp.float32)
            a1 = jnp.zeros((16,), jnp.float32)
            for c in range(8):
                w = w_buf[c, pl.ds(g * 16, 16)]
                r = rows_buf[c, pl.ds(g * 16, 16)]
                f0 = plsc.bitcast(lax.shift_left(r, 16), jnp.float32)
                f1 = plsc.bitcast(r & _HI, jnp.float32)
                a0 = a0 + w * f0
                a1 = a1 + w * f1
            plsc.store_scatter(out_buf, [pv, col0], a0)
            plsc.store_scatter(out_buf, [pv, col1], a1)
            return c2

        lax.fori_loop(0, _NG, acc, 0)

    def chunk_body(ch, carry):
        cbase = ch * _C
        pend = make_comp(cbase, 0)
        for l in range(1, _NUM_LEVELS):
            nxt = make_comp(cbase, l)
            drain_acc(pend, l - 1)
            pend = nxt
        drain_acc(pend, _NUM_LEVELS - 1)
        pltpu.sync_copy(out_buf, out_hbm.at[pl.ds(base + cbase, _C), :])
        return carry

    lax.fori_loop(0, _NCH, chunk_body, 0)


def kernel(inputs, embeddings):
    packed = lax.bitcast_convert_type(
        embeddings.astype(jnp.bfloat16), jnp.int32)
    return _encode(inputs, packed)


# speedup vs baseline: 6.5717x; 1.3793x over previous
import functools

import jax
import jax.numpy as jnp
import numpy as np
from jax import lax
from jax.experimental import pallas as pl
from jax.experimental.pallas import tpu as pltpu
from jax.experimental.pallas import tpu_sc as plsc

_INPUT_DIM = 3
_NUM_LEVELS = 16
_LEVEL_DIM = 2
_BASE_RES = 16
_MAX_PARAMS = 2 ** 19
_B = 262144
_OUT_DIM = _NUM_LEVELS * _LEVEL_DIM

_P1 = np.int32(np.int64(2654435761) - (1 << 32))
_P2 = np.int32(805459861)

_NC, _NS = 2, 16
_NW = _NC * _NS
_PPW = _B // _NW
_C = 128
_NCH = _PPW // _C
_NG = _C // 16


def _level_offsets():
    offs, o = [0], 0
    for i in range(_NUM_LEVELS):
        res = _BASE_RES * (2 ** i)
        o += min(_MAX_PARAMS, (res + 1) ** _INPUT_DIM)
        offs.append(o)
    return offs


_OFFS = _level_offsets()
_TABLE_ROWS = _OFFS[-1]
_NLOCAL = 2
_LTAB = _OFFS[_NLOCAL]
_LTAB_PAD = -(-_LTAB // 8) * 8
_SEG = 2048

_mesh = plsc.VectorSubcoreMesh(core_axis_name="c", subcore_axis_name="s")


@functools.partial(
    pl.kernel,
    out_type=jax.ShapeDtypeStruct((_B, _OUT_DIM), jnp.float32),
    mesh=_mesh,
    compiler_params=pltpu.CompilerParams(needs_layout_passes=False,
                                         use_tc_tiling_on_sc=False),
    scratch_types=[
        pltpu.VMEM((_SEG, _INPUT_DIM), jnp.float32),
        pltpu.VMEM((_INPUT_DIM * _PPW,), jnp.float32),
        pltpu.VMEM((8, _C), jnp.int32),
        pltpu.VMEM((8, _C), jnp.int32),
        pltpu.VMEM((8, _C), jnp.float32),
        pltpu.VMEM((8, _C), jnp.float32),
        pltpu.VMEM((8, _C), jnp.int32),
        pltpu.VMEM((8, _C), jnp.int32),
        pltpu.VMEM((_C, _OUT_DIM), jnp.float32),
        pltpu.VMEM((_LTAB_PAD,), jnp.int32),
        pltpu.SemaphoreType.DMA,
        pltpu.SemaphoreType.DMA,
    ],
)
def _encode(in_hbm, emb_hbm, out_hbm, xin, xs, idx0, idx1,
            w0, w1, rows0, rows1, out_buf, ltab, semA, semB):
    idxb, wb, rowsb, sems = ((idx0, idx1), (w0, w1), (rows0, rows1),
                             (semA, semB))

    cid = lax.axis_index("c")
    sid = lax.axis_index("s")
    wid = sid * _NC + cid
    base = wid * _PPW

    pltpu.sync_copy(emb_hbm.at[pl.ds(0, _LTAB_PAD)], ltab)

    iota = lax.iota(jnp.int32, 16)

    for seg in range(_PPW // _SEG):
        pltpu.sync_copy(in_hbm.at[pl.ds(base + seg * _SEG, _SEG), :], xin)

        def deint(g, carry, seg=seg):
            pv = g * 16 + iota
            for d in range(_INPUT_DIM):
                v = plsc.load_gather(xin, [pv, jnp.full((16,), d, jnp.int32)])
                x = jnp.clip((v + 1.0) * 0.5, 0.0, 1.0)
                xs[pl.ds(d * _PPW + seg * _SEG + g * 16, 16)] = x
            return carry

        lax.fori_loop(0, _SEG // 16, deint, 0)

    def level_const(l):
        res = _BASE_RES * (2 ** l)
        ts = _OFFS[l + 1] - _OFFS[l]
        return res, ts, _OFFS[l], (res + 1) ** _INPUT_DIM > ts

    def make_comp(cbase, l):
        res, ts, off, use_hash = level_const(l)
        resf = float(res)
        par = l % 2
        idx_buf, w_buf = idxb[par], wb[par]

        def comp(g, c2):
            p0 = cbase + g * 16
            ii, fr = [], []
            for d in range(_INPUT_DIM):
                x = xs[pl.ds(d * _PPW + p0, 16)]
                pos = x * resf
                i = jnp.minimum(pos.astype(jnp.int32), jnp.int32(res - 1))
                ii.append(i)
                fr.append(pos - i.astype(jnp.float32))
            if use_hash:
                lo = [ii[0], ii[1] * _P1, ii[2] * _P2]
                hi = [ii[0] + 1, (ii[1] + 1) * _P1, (ii[2] + 1) * _P2]
            else:
                s1, s2 = res + 1, (res + 1) ** 2
                lo = [ii[0], ii[1] * jnp.int32(s1), ii[2] * jnp.int32(s2)]
                hi = [ii[0] + 1, lo[1] + s1, lo[2] + s2]
            t = [1.0 - fr[d] for d in range(_INPUT_DIM)]
            q = [t[0] * t[1], fr[0] * t[1], t[0] * fr[1], fr[0] * fr[1]]
            for c in range(8):
                b0, b1, b2 = c & 1, (c >> 1) & 1, (c >> 2) & 1
                e0 = hi[0] if b0 else lo[0]
                e1 = hi[1] if b1 else lo[1]
                e2 = hi[2] if b2 else lo[2]
                if use_hash:
                    idx = ((e0 ^ e1) ^ e2) & jnp.int32(ts - 1)
                else:
                    idx = e0 + e1 + e2
                idx_buf[c, pl.ds(g * 16, 16)] = idx + jnp.int32(off)
                w = q[b0 + 2 * b1] * (fr[2] if b2 else t[2])
                w_buf[c, pl.ds(g * 16, 16)] = w
            return c2

        lax.fori_loop(0, _NG, comp, 0)
        if l < _NLOCAL:
            return []
        par_sem = sems[par]
        return [pltpu.async_copy(emb_hbm.at[idxb[par].at[c]],
                                 rowsb[par].at[c], par_sem)
                for c in range(8)]

    _HI = np.int32(np.int64(0xFFFF0000) - (1 << 32))

    def drain_acc(cps, l):
        for cp in cps:
            cp.wait()
        par = l % 2
        idx_buf, w_buf, rows_buf = idxb[par], wb[par], rowsb[par]
        local = l < _NLOCAL
        col0 = jnp.full((16,), 2 * l, jnp.int32)
        col1 = jnp.full((16,), 2 * l + 1, jnp.int32)

        def acc(g, c2):
            pv = g * 16 + iota
            a0 = jnp.zeros((16,), jnp.float32)
            a1 = jnp.zeros((16,), jnp.float32)
            for c in range(8):
                w = w_buf[c, pl.ds(g * 16, 16)]
                if local:
                    r = plsc.load_gather(ltab,
                                         [idx_buf[c, pl.ds(g * 16, 16)]])
                else:
                    r = rows_buf[c, pl.ds(g * 16, 16)]
                f0 = plsc.bitcast(lax.shift_left(r, 16), jnp.float32)
                f1 = plsc.bitcast(r & _HI, jnp.float32)
                a0 = a0 + w * f0
                a1 = a1 + w * f1
            plsc.store_scatter(out_buf, [pv, col0], a0)
            plsc.store_scatter(out_buf, [pv, col1], a1)
            return c2

        lax.fori_loop(0, _NG, acc, 0)

    def chunk_body(ch, carry):
        cbase = ch * _C
        pend = make_comp(cbase, 0)
        for l in range(1, _NUM_LEVELS):
            nxt = make_comp(cbase, l)
            drain_acc(pend, l - 1)
            pend = nxt
        drain_acc(pend, _NUM_LEVELS - 1)
        pltpu.sync_copy(out_buf, out_hbm.at[pl.ds(base + cbase, _C), :])
        return carry

    lax.fori_loop(0, _NCH, chunk_body, 0)


def kernel(inputs, embeddings):
    packed = lax.bitcast_convert_type(
        embeddings.astype(jnp.bfloat16), jnp.int32)
    return _encode(inputs, packed)
